# baseline (speedup 1.0000x reference)
import functools
import math

import jax
import jax.numpy as jnp
from jax import lax
from jax.experimental import pallas as pl
from jax.experimental.pallas import tpu as pltpu
from jax.experimental.pallas import tpu_sc as plsc

L = 16


def _signed_key(b):
    return b ^ (lax.shift_right_arithmetic(b, 31) & jnp.int32(0x7FFFFFFF))


def _matmul_t_body(x_ref, w_ref, o_ref):
    o_ref[...] = lax.dot_general(
        w_ref[...], x_ref[...], (((1,), (1,)), ((), ())),
        preferred_element_type=jnp.float32)


def _scores_t(x, w):
    B, D = x.shape
    O = w.shape[0]
    CB = 128
    return pl.pallas_call(
        _matmul_t_body,
        grid=(O // CB,),
        in_specs=[
            pl.BlockSpec((B, D), lambda j: (0, 0)),
            pl.BlockSpec((CB, D), lambda j: (j, 0)),
        ],
        out_specs=pl.BlockSpec((CB, B), lambda j: (j, 0)),
        out_shape=jax.ShapeDtypeStruct((O, B), jnp.float32),
    )(x, w)


def _sc_select_body(k_active, n, scores_hbm, thr_hbm,
                    data_a, data_b, lista_v, hist_v, totals_v, gsum_v, thr_v,
                    sem_a, sem_b):
    nv = n // L
    lane = lax.iota(jnp.int32, L)
    ones = jnp.ones((L,), jnp.int32)
    zeros = jnp.zeros((L,), jnp.int32)
    c = lax.axis_index("c")
    s = lax.axis_index("s")
    wid = s * 2 + c
    rpw = 32
    row0 = wid * rpw

    lane256 = lane * 256
    mmin = jnp.int32(-2147483648)
    rank0 = jnp.full((L,), k_active, jnp.int32)

    def take(vec, idx):
        return jnp.sum(jnp.where(lane == idx, vec, 0))

    def level(counts, rank):
        rT = lax.rev(counts, (0,))
        cT = plsc.cumsum(rT)
        m = cT >= rank
        pc = plsc.all_reduce_population_count(m)
        f = jnp.int32(16) - pc
        above = take(cT - rT, f)
        return pc - 1, rank - above

    def select_row(data_v, r):
        @plsc.parallel_loop(0, nv, unroll=8)
        def l1(i):
            b = plsc.bitcast(plsc.load_gather(data_v, [i * L + lane]), jnp.int32)
            d = lax.shift_right_logical(b, 24)
            plsc.addupdate_scatter(hist_v, [lane256 | d], ones)

        @plsc.parallel_loop(0, 16, unroll=2)
        def red(g):
            acc = zeros
            for l in range(L):
                acc = acc + plsc.load_gather(hist_v, [l * 256 + g * L + lane])
            raw = g * L + lane
            flip = jnp.where(raw < 128, jnp.int32(0x80), jnp.int32(0xFF))
            plsc.store_scatter(totals_v, [raw ^ flip], acc)
            ukg = jnp.where(g < 8, g + 8, 15 - g)
            gtot = jnp.full((L,), jnp.sum(acc), jnp.int32)
            plsc.store_scatter(gsum_v, [jnp.full((L,), ukg, jnp.int32)],
                               gtot, mask=lane == 0)

        gs = gsum_v[...]
        gstar, rank_g = level(gs, rank0)
        T = plsc.load_gather(totals_v, [gstar * L + lane])
        dsub, rank2 = level(T, rank_g)
        d1 = gstar * L + dsub
        d1_raw = d1 ^ jnp.where(d1 >= 128, jnp.int32(0x80), jnp.int32(0xFF))

        cnts = plsc.load_gather(hist_v, [lane256 | d1_raw])
        base = plsc.cumsum(cnts) - cnts
        n1 = jnp.sum(cnts)

        @plsc.parallel_loop(0, 256, unroll=8)
        def zb(i):
            plsc.store_scatter(hist_v, [lane256 | i], zeros)

        @plsc.parallel_loop(0, nv, unroll=8, carry=base)
        def l2(i, off):
            b = plsc.bitcast(plsc.load_gather(data_v, [i * L + lane]), jnp.int32)
            m = lax.shift_right_logical(b, 24) == d1_raw
            val = b ^ lax.shift_right_arithmetic(b, 31)
            plsc.store_scatter(lista_v, [off], val, mask=m)
            return off + m.astype(jnp.int32)

        nv1 = (n1 + (L - 1)) // L

        def bit_step(bi, t):
            cand = t | lax.shift_left(jnp.int32(1), jnp.int32(23) - bi)

            @plsc.parallel_loop(0, nv1, carry=zeros)
            def cnt_loop(i, cntv):
                kv = plsc.load_gather(lista_v, [i * L + lane])
                low = kv & jnp.int32(0x00FFFFFF)
                ok = jnp.logical_and(low >= cand, i * L + lane < n1)
                return cntv + plsc.all_reduce_population_count(ok)
            return jnp.where(cnt_loop >= rank2, cand, t)

        t_low = lax.fori_loop(0, 24, bit_step, jnp.full((L,), 0, jnp.int32))
        kth = lax.shift_left(d1, 24) | t_low
        kth_s = kth ^ mmin

        plsc.store_scatter(thr_v, [jnp.full((L,), r, jnp.int32)],
                           kth_s, mask=lane == 0)

    @plsc.parallel_loop(0, 256, unroll=8)
    def zb0(i):
        plsc.store_scatter(hist_v, [lane256 | i], zeros)

    pltpu.make_async_copy(scores_hbm.at[row0], data_a, sem_a).start()
    pltpu.make_async_copy(scores_hbm.at[row0 + 1], data_b, sem_b).start()

    def do_pair(m, u):
        r = 2 * m
        pltpu.make_async_copy(scores_hbm.at[row0], data_a, sem_a).wait()
        select_row(data_a, r)
        nxt = row0 + lax.rem(r + 2, rpw)
        pltpu.make_async_copy(scores_hbm.at[nxt], data_a, sem_a).start()
        pltpu.make_async_copy(scores_hbm.at[row0 + 1], data_b, sem_b).wait()
        select_row(data_b, r + 1)
        nxt2 = row0 + lax.rem(r + 3, rpw)
        pltpu.make_async_copy(scores_hbm.at[nxt2], data_b, sem_b).start()
        return u
    lax.fori_loop(0, rpw // 2, do_pair, 0)

    pltpu.make_async_copy(scores_hbm.at[row0], data_a, sem_a).wait()
    pltpu.make_async_copy(scores_hbm.at[row0 + 1], data_b, sem_b).wait()

    pltpu.sync_copy(thr_v, thr_hbm.at[pl.ds(row0, rpw)])


def _select_thresholds(scores_t, k_active):
    O, B = scores_t.shape
    mesh = plsc.VectorSubcoreMesh(core_axis_name="c", subcore_axis_name="s")
    f = pl.kernel(
        functools.partial(_sc_select_body, k_active, B),
        mesh=mesh,
        compiler_params=pltpu.CompilerParams(needs_layout_passes=False),
        out_type=jax.ShapeDtypeStruct((O,), jnp.int32),
        scratch_types=[
            pltpu.VMEM((B,), jnp.float32),
            pltpu.VMEM((B,), jnp.float32),
            pltpu.VMEM((B,), jnp.int32),
            pltpu.VMEM((L * 256,), jnp.int32),
            pltpu.VMEM((256,), jnp.int32),
            pltpu.VMEM((16,), jnp.int32),
            pltpu.VMEM((32,), jnp.int32),
            pltpu.SemaphoreType.DMA,
            pltpu.SemaphoreType.DMA,
        ],
    )
    return f(scores_t)


def _mask_body(x_ref, w_ref, b_ref, t_ref, o_ref):
    x = x_ref[...]
    w = w_ref[...]
    sc = lax.dot_general(
        x, w, (((1,), (1,)), ((), ())), preferred_element_type=jnp.float32)
    ks = _signed_key(lax.bitcast_convert_type(sc, jnp.int32))
    keep = ks >= t_ref[...]
    o_ref[...] = jnp.where(keep, sc + b_ref[...], jnp.float32(0.0))


def _masked_out(x, w, bias2, thr2):
    B, D = x.shape
    O = w.shape[0]
    CB = 128
    return pl.pallas_call(
        _mask_body,
        grid=(O // CB,),
        in_specs=[
            pl.BlockSpec((B, D), lambda j: (0, 0)),
            pl.BlockSpec((CB, D), lambda j: (j, 0)),
            pl.BlockSpec((1, CB), lambda j: (0, j)),
            pl.BlockSpec((1, CB), lambda j: (0, j)),
        ],
        out_specs=pl.BlockSpec((B, CB), lambda j: (0, j)),
        out_shape=jax.ShapeDtypeStruct((B, O), jnp.float32),
    )(x, w, bias2, thr2)


@jax.jit
def kernel(input, weight, bias):
    B, D = input.shape
    O = weight.shape[0]
    k_active = math.ceil(0.5 * B)
    st = _scores_t(input, weight)
    thr = _select_thresholds(st, k_active)
    return _masked_out(input, weight, bias.reshape(1, O), thr.reshape(1, O))


# speedup vs baseline: 1.0052x; 1.0052x over previous
import functools
import math

import jax
import jax.numpy as jnp
from jax import lax
from jax.experimental import pallas as pl
from jax.experimental.pallas import tpu as pltpu
from jax.experimental.pallas import tpu_sc as plsc

L = 16


def _signed_key(b):
    return b ^ (lax.shift_right_arithmetic(b, 31) & jnp.int32(0x7FFFFFFF))


def _matmul_t_body(x_ref, w_ref, o_ref):
    o_ref[...] = lax.dot_general(
        w_ref[...], x_ref[...], (((1,), (1,)), ((), ())),
        preferred_element_type=jnp.float32)


def _scores_t(x, w):
    B, D = x.shape
    O = w.shape[0]
    CB = 128
    return pl.pallas_call(
        _matmul_t_body,
        grid=(O // CB,),
        in_specs=[
            pl.BlockSpec((B, D), lambda j: (0, 0)),
            pl.BlockSpec((CB, D), lambda j: (j, 0)),
        ],
        out_specs=pl.BlockSpec((CB, B), lambda j: (j, 0)),
        out_shape=jax.ShapeDtypeStruct((O, B), jnp.float32),
    )(x, w)


def _sc_select_body(k_active, n, scores_hbm, thr_hbm,
                    data_a, data_b, lista_v, hist_v, totals_v, gsum_v, thr_v,
                    sem_a, sem_b):
    nv = n // L
    lane = lax.iota(jnp.int32, L)
    ones = jnp.ones((L,), jnp.int32)
    zeros = jnp.zeros((L,), jnp.int32)
    c = lax.axis_index("c")
    s = lax.axis_index("s")
    wid = s * 2 + c
    rpw = 32
    row0 = wid * rpw

    lane256 = lane * 256
    mmin = jnp.int32(-2147483648)
    rank0 = jnp.full((L,), k_active, jnp.int32)

    def take(vec, idx):
        return jnp.sum(jnp.where(lane == idx, vec, 0))

    def level(counts, rank):
        rT = lax.rev(counts, (0,))
        cT = plsc.cumsum(rT)
        m = cT >= rank
        pc = plsc.all_reduce_population_count(m)
        f = jnp.int32(16) - pc
        above = take(cT - rT, f)
        return pc - 1, rank - above

    def select_row(data_v, r):
        @plsc.parallel_loop(0, nv, unroll=8)
        def l1(i):
            b = plsc.bitcast(plsc.load_gather(data_v, [i * L + lane]), jnp.int32)
            d = lax.shift_right_logical(b, 24)
            plsc.addupdate_scatter(hist_v, [lane256 | d], ones)

        @plsc.parallel_loop(0, 16, unroll=2)
        def red(g):
            acc = zeros
            for l in range(L):
                acc = acc + plsc.load_gather(hist_v, [l * 256 + g * L + lane])
            raw = g * L + lane
            flip = jnp.where(raw < 128, jnp.int32(0x80), jnp.int32(0xFF))
            plsc.store_scatter(totals_v, [raw ^ flip], acc)
            ukg = jnp.where(g < 8, g + 8, 15 - g)
            gtot = jnp.full((L,), jnp.sum(acc), jnp.int32)
            plsc.store_scatter(gsum_v, [jnp.full((L,), ukg, jnp.int32)],
                               gtot, mask=lane == 0)

        gs = gsum_v[...]
        gstar, rank_g = level(gs, rank0)
        T = plsc.load_gather(totals_v, [gstar * L + lane])
        dsub, rank2 = level(T, rank_g)
        d1 = gstar * L + dsub
        d1_raw = d1 ^ jnp.where(d1 >= 128, jnp.int32(0x80), jnp.int32(0xFF))

        cnts = plsc.load_gather(hist_v, [lane256 | d1_raw])
        base = plsc.cumsum(cnts) - cnts
        n1 = jnp.sum(cnts)

        @plsc.parallel_loop(0, 256, unroll=8)
        def zb(i):
            plsc.store_scatter(hist_v, [lane256 | i], zeros)

        d1_sr = jnp.where(d1_raw >= 128, jnp.int32(-1), jnp.int32(0))

        @plsc.parallel_loop(0, nv, unroll=8, carry=base)
        def l2(i, off):
            b = plsc.bitcast(plsc.load_gather(data_v, [i * L + lane]), jnp.int32)
            m = lax.shift_right_logical(b, 24) == d1_raw
            val = b ^ d1_sr
            plsc.store_scatter(lista_v, [off], val, mask=m)
            return off + m.astype(jnp.int32)

        nv1 = (n1 + (L - 1)) // L

        def bit_step(bi, t):
            cand = t | lax.shift_left(jnp.int32(1), jnp.int32(23) - bi)

            @plsc.parallel_loop(0, nv1, carry=zeros)
            def cnt_loop(i, cntv):
                kv = plsc.load_gather(lista_v, [i * L + lane])
                low = kv & jnp.int32(0x00FFFFFF)
                ok = jnp.logical_and(low >= cand, i * L + lane < n1)
                return cntv + ok.astype(jnp.int32)
            return jnp.where(jnp.sum(cnt_loop) >= rank2, cand, t)

        t_low = lax.fori_loop(0, 24, bit_step, jnp.full((L,), 0, jnp.int32))
        kth = lax.shift_left(d1, 24) | t_low
        kth_s = kth ^ mmin

        plsc.store_scatter(thr_v, [jnp.full((L,), r, jnp.int32)],
                           kth_s, mask=lane == 0)

    @plsc.parallel_loop(0, 256, unroll=8)
    def zb0(i):
        plsc.store_scatter(hist_v, [lane256 | i], zeros)

    pltpu.make_async_copy(scores_hbm.at[row0], data_a, sem_a).start()
    pltpu.make_async_copy(scores_hbm.at[row0 + 1], data_b, sem_b).start()

    def do_pair(m, u):
        r = 2 * m
        pltpu.make_async_copy(scores_hbm.at[row0], data_a, sem_a).wait()
        select_row(data_a, r)
        nxt = row0 + lax.rem(r + 2, rpw)
        pltpu.make_async_copy(scores_hbm.at[nxt], data_a, sem_a).start()
        pltpu.make_async_copy(scores_hbm.at[row0 + 1], data_b, sem_b).wait()
        select_row(data_b, r + 1)
        nxt2 = row0 + lax.rem(r + 3, rpw)
        pltpu.make_async_copy(scores_hbm.at[nxt2], data_b, sem_b).start()
        return u
    lax.fori_loop(0, rpw // 2, do_pair, 0)

    pltpu.make_async_copy(scores_hbm.at[row0], data_a, sem_a).wait()
    pltpu.make_async_copy(scores_hbm.at[row0 + 1], data_b, sem_b).wait()

    pltpu.sync_copy(thr_v, thr_hbm.at[pl.ds(row0, rpw)])


def _select_thresholds(scores_t, k_active):
    O, B = scores_t.shape
    mesh = plsc.VectorSubcoreMesh(core_axis_name="c", subcore_axis_name="s")
    f = pl.kernel(
        functools.partial(_sc_select_body, k_active, B),
        mesh=mesh,
        compiler_params=pltpu.CompilerParams(needs_layout_passes=False),
        out_type=jax.ShapeDtypeStruct((O,), jnp.int32),
        scratch_types=[
            pltpu.VMEM((B,), jnp.float32),
            pltpu.VMEM((B,), jnp.float32),
            pltpu.VMEM((B,), jnp.int32),
            pltpu.VMEM((L * 256,), jnp.int32),
            pltpu.VMEM((256,), jnp.int32),
            pltpu.VMEM((16,), jnp.int32),
            pltpu.VMEM((32,), jnp.int32),
            pltpu.SemaphoreType.DMA,
            pltpu.SemaphoreType.DMA,
        ],
    )
    return f(scores_t)


def _mask_body(x_ref, w_ref, b_ref, t_ref, o_ref):
    x = x_ref[...]
    w = w_ref[...]
    sc = lax.dot_general(
        x, w, (((1,), (1,)), ((), ())), preferred_element_type=jnp.float32)
    ks = _signed_key(lax.bitcast_convert_type(sc, jnp.int32))
    keep = ks >= t_ref[...]
    o_ref[...] = jnp.where(keep, sc + b_ref[...], jnp.float32(0.0))


def _masked_out(x, w, bias2, thr2):
    B, D = x.shape
    O = w.shape[0]
    CB = 128
    return pl.pallas_call(
        _mask_body,
        grid=(O // CB,),
        in_specs=[
            pl.BlockSpec((B, D), lambda j: (0, 0)),
            pl.BlockSpec((CB, D), lambda j: (j, 0)),
            pl.BlockSpec((1, CB), lambda j: (0, j)),
            pl.BlockSpec((1, CB), lambda j: (0, j)),
        ],
        out_specs=pl.BlockSpec((B, CB), lambda j: (0, j)),
        out_shape=jax.ShapeDtypeStruct((B, O), jnp.float32),
    )(x, w, bias2, thr2)


@jax.jit
def kernel(input, weight, bias):
    B, D = input.shape
    O = weight.shape[0]
    k_active = math.ceil(0.5 * B)
    st = _scores_t(input, weight)
    thr = _select_thresholds(st, k_active)
    return _masked_out(input, weight, bias.reshape(1, O), thr.reshape(1, O))


# speedup vs baseline: 1.4118x; 1.4046x over previous
import functools
import math

import jax
import jax.numpy as jnp
from jax import lax
from jax.experimental import pallas as pl
from jax.experimental.pallas import tpu as pltpu
from jax.experimental.pallas import tpu_sc as plsc

L = 16


def _signed_key(b):
    return b ^ (lax.shift_right_arithmetic(b, 31) & jnp.int32(0x7FFFFFFF))


def _matmul_t_body(x_ref, w_ref, o_ref):
    o_ref[...] = lax.dot_general(
        w_ref[...], x_ref[...], (((1,), (1,)), ((), ())),
        preferred_element_type=jnp.float32)


def _scores_t(x, w):
    B, D = x.shape
    O = w.shape[0]
    CB = 128
    return pl.pallas_call(
        _matmul_t_body,
        grid=(O // CB,),
        in_specs=[
            pl.BlockSpec((B, D), lambda j: (0, 0)),
            pl.BlockSpec((CB, D), lambda j: (j, 0)),
        ],
        out_specs=pl.BlockSpec((CB, B), lambda j: (j, 0)),
        out_shape=jax.ShapeDtypeStruct((O, B), jnp.float32),
    )(x, w)


def _sc_select_body(k_active, n, scores_hbm, thr_hbm,
                    data_a, data_b, lista_v, hist_v, totals_v, gsum_v, thr_v,
                    sem_a, sem_b):
    nv = n // L
    lane = lax.iota(jnp.int32, L)
    ones = jnp.ones((L,), jnp.int32)
    zeros = jnp.zeros((L,), jnp.int32)
    c = lax.axis_index("c")
    s = lax.axis_index("s")
    wid = s * 2 + c
    rpw = 32
    row0 = wid * rpw

    lane256 = lane * 256
    mmin = jnp.int32(-2147483648)
    rank0 = jnp.full((L,), k_active, jnp.int32)

    def take(vec, idx):
        return jnp.sum(jnp.where(lane == idx, vec, 0))

    def level(counts, rank):
        rT = lax.rev(counts, (0,))
        cT = plsc.cumsum(rT)
        m = cT >= rank
        pc = plsc.all_reduce_population_count(m)
        f = jnp.int32(16) - pc
        above = take(cT - rT, f)
        return pc - 1, rank - above

    def select_row(data_v, r):
        @plsc.parallel_loop(0, nv, unroll=8)
        def l1(i):
            b = plsc.bitcast(plsc.load_gather(data_v, [i * L + lane]), jnp.int32)
            d = lax.shift_right_logical(b, 24)
            plsc.addupdate_scatter(hist_v, [lane256 | (d ^ lane)], ones)

        @plsc.parallel_loop(0, 16, unroll=2)
        def red(g):
            acc = zeros
            raw = g * L + lane
            for l in range(L):
                acc = acc + plsc.load_gather(hist_v, [l * 256 + (raw ^ l)])
            flip = jnp.where(raw < 128, jnp.int32(0x80), jnp.int32(0xFF))
            plsc.store_scatter(totals_v, [raw ^ flip], acc)
            ukg = jnp.where(g < 8, g + 8, 15 - g)
            gtot = jnp.full((L,), jnp.sum(acc), jnp.int32)
            plsc.store_scatter(gsum_v, [jnp.full((L,), ukg, jnp.int32)],
                               gtot, mask=lane == 0)

        gs = gsum_v[...]
        gstar, rank_g = level(gs, rank0)
        T = plsc.load_gather(totals_v, [gstar * L + lane])
        dsub, rank2 = level(T, rank_g)
        d1 = gstar * L + dsub
        d1_raw = d1 ^ jnp.where(d1 >= 128, jnp.int32(0x80), jnp.int32(0xFF))

        cnts = plsc.load_gather(hist_v, [lane256 | (d1_raw ^ lane)])
        base = plsc.cumsum(cnts) - cnts
        n1 = jnp.sum(cnts)

        @plsc.parallel_loop(0, 256, unroll=8)
        def zb(i):
            plsc.store_scatter(hist_v, [lane256 | i], zeros)

        d1_sr = jnp.where(d1_raw >= 128, jnp.int32(-1), jnp.int32(0))

        @plsc.parallel_loop(0, nv, unroll=8, carry=base)
        def l2(i, off):
            b = plsc.bitcast(plsc.load_gather(data_v, [i * L + lane]), jnp.int32)
            m = lax.shift_right_logical(b, 24) == d1_raw
            val = b ^ d1_sr
            plsc.store_scatter(lista_v, [off], val, mask=m)
            return off + m.astype(jnp.int32)

        nv1 = (n1 + (L - 1)) // L

        def bit_step(bi, t):
            cand = t | lax.shift_left(jnp.int32(1), jnp.int32(23) - bi)

            @plsc.parallel_loop(0, nv1, carry=zeros)
            def cnt_loop(i, cntv):
                kv = plsc.load_gather(lista_v, [i * L + lane])
                low = kv & jnp.int32(0x00FFFFFF)
                ok = jnp.logical_and(low >= cand, i * L + lane < n1)
                return cntv + ok.astype(jnp.int32)
            return jnp.where(jnp.sum(cnt_loop) >= rank2, cand, t)

        t_low = lax.fori_loop(0, 24, bit_step, jnp.full((L,), 0, jnp.int32))
        kth = lax.shift_left(d1, 24) | t_low
        kth_s = kth ^ mmin

        plsc.store_scatter(thr_v, [jnp.full((L,), r, jnp.int32)],
                           kth_s, mask=lane == 0)

    @plsc.parallel_loop(0, 256, unroll=8)
    def zb0(i):
        plsc.store_scatter(hist_v, [lane256 | i], zeros)

    pltpu.make_async_copy(scores_hbm.at[row0], data_a, sem_a).start()
    pltpu.make_async_copy(scores_hbm.at[row0 + 1], data_b, sem_b).start()

    def do_pair(m, u):
        r = 2 * m
        pltpu.make_async_copy(scores_hbm.at[row0], data_a, sem_a).wait()
        select_row(data_a, r)
        nxt = row0 + lax.rem(r + 2, rpw)
        pltpu.make_async_copy(scores_hbm.at[nxt], data_a, sem_a).start()
        pltpu.make_async_copy(scores_hbm.at[row0 + 1], data_b, sem_b).wait()
        select_row(data_b, r + 1)
        nxt2 = row0 + lax.rem(r + 3, rpw)
        pltpu.make_async_copy(scores_hbm.at[nxt2], data_b, sem_b).start()
        return u
    lax.fori_loop(0, rpw // 2, do_pair, 0)

    pltpu.make_async_copy(scores_hbm.at[row0], data_a, sem_a).wait()
    pltpu.make_async_copy(scores_hbm.at[row0 + 1], data_b, sem_b).wait()

    pltpu.sync_copy(thr_v, thr_hbm.at[pl.ds(row0, rpw)])


def _select_thresholds(scores_t, k_active):
    O, B = scores_t.shape
    mesh = plsc.VectorSubcoreMesh(core_axis_name="c", subcore_axis_name="s")
    f = pl.kernel(
        functools.partial(_sc_select_body, k_active, B),
        mesh=mesh,
        compiler_params=pltpu.CompilerParams(needs_layout_passes=False),
        out_type=jax.ShapeDtypeStruct((O,), jnp.int32),
        scratch_types=[
            pltpu.VMEM((B,), jnp.float32),
            pltpu.VMEM((B,), jnp.float32),
            pltpu.VMEM((B,), jnp.int32),
            pltpu.VMEM((L * 256,), jnp.int32),
            pltpu.VMEM((256,), jnp.int32),
            pltpu.VMEM((16,), jnp.int32),
            pltpu.VMEM((32,), jnp.int32),
            pltpu.SemaphoreType.DMA,
            pltpu.SemaphoreType.DMA,
        ],
    )
    return f(scores_t)


def _mask_body(x_ref, w_ref, b_ref, t_ref, o_ref):
    x = x_ref[...]
    w = w_ref[...]
    sc = lax.dot_general(
        x, w, (((1,), (1,)), ((), ())), preferred_element_type=jnp.float32)
    ks = _signed_key(lax.bitcast_convert_type(sc, jnp.int32))
    keep = ks >= t_ref[...]
    o_ref[...] = jnp.where(keep, sc + b_ref[...], jnp.float32(0.0))


def _masked_out(x, w, bias2, thr2):
    B, D = x.shape
    O = w.shape[0]
    CB = 128
    return pl.pallas_call(
        _mask_body,
        grid=(O // CB,),
        in_specs=[
            pl.BlockSpec((B, D), lambda j: (0, 0)),
            pl.BlockSpec((CB, D), lambda j: (j, 0)),
            pl.BlockSpec((1, CB), lambda j: (0, j)),
            pl.BlockSpec((1, CB), lambda j: (0, j)),
        ],
        out_specs=pl.BlockSpec((B, CB), lambda j: (0, j)),
        out_shape=jax.ShapeDtypeStruct((B, O), jnp.float32),
    )(x, w, bias2, thr2)


@jax.jit
def kernel(input, weight, bias):
    B, D = input.shape
    O = weight.shape[0]
    k_active = math.ceil(0.5 * B)
    st = _scores_t(input, weight)
    thr = _select_thresholds(st, k_active)
    return _masked_out(input, weight, bias.reshape(1, O), thr.reshape(1, O))
